# baseline (speedup 1.0000x reference)
import dataclasses
import functools

import jax
import jax.numpy as jnp
from jax import lax
from jax.experimental import pallas as pl
from jax.experimental.pallas import tpu as pltpu
from jax.experimental.pallas import tpu_sc as plsc

N = 10000
E = 160000
D_OBJ = 256
D_PRED = 128
H = 512
D_OUT = 256
D_OUT_PRED = 128

NPAD = 10240
RPS = NPAD // 16
WIN = 128
BE = 640
BN = 512
SW = 128
CCH = 2000
CHUNKS = (20480, 49920, 49920, 39680)

_mesh = plsc.VectorSubcoreMesh(core_axis_name="c", subcore_axis_name="s")

_sc_params = pltpu.CompilerParams()
if "needs_layout_passes" in pltpu.CompilerParams.__dataclass_fields__:
    _sc_params = dataclasses.replace(_sc_params, needs_layout_passes=False)


@functools.lru_cache(maxsize=None)
def _make_gather(ne):
    nwin = ne // WIN

    @functools.partial(
        pl.kernel,
        mesh=_mesh,
        out_type=[
            jax.ShapeDtypeStruct((ne, D_OBJ), jnp.float32),
            jax.ShapeDtypeStruct((ne, D_OBJ), jnp.float32),
        ],
        scratch_types=[
            pltpu.VMEM((WIN,), jnp.int32),
            pltpu.VMEM((WIN,), jnp.int32),
            pltpu.VMEM((WIN, D_OBJ), jnp.float32),
            pltpu.VMEM((WIN, D_OBJ), jnp.float32),
            pltpu.SemaphoreType.DMA,
            pltpu.SemaphoreType.DMA,
            pltpu.SemaphoreType.DMA,
            pltpu.SemaphoreType.DMA,
        ],
    )
    def gather(obj_h, sidx_h, oidx_h, gs_h, go_h,
               idx0, idx1, rows0, rows1, si0, si1, sg0, sg1):
        wid = lax.axis_index("s") * 2 + lax.axis_index("c")
        full = nwin // 32
        rem = nwin % 32

        def one_pass(eidx_h, out_h):
            def icp(k, idxb, sem):
                return pltpu.make_async_copy(
                    eidx_h.at[pl.ds((k * 32 + wid) * WIN, WIN)], idxb, sem)

            def gcp(idxb, rowsb, sem):
                return pltpu.make_async_copy(obj_h.at[idxb], rowsb, sem)

            def wb(k, rowsb):
                pltpu.sync_copy(
                    rowsb, out_h.at[pl.ds((k * 32 + wid) * WIN, WIN)])

            icp(0, idx0, si0).start()
            icp(0, idx0, si0).wait()
            gcp(idx0, rows0, sg0).start()
            icp(1, idx1, si1).start()

            @pl.loop(0, full // 2)
            def _(t):
                a = 2 * t
                icp(a + 1, idx1, si1).wait()
                gcp(idx1, rows1, sg1).start()

                @pl.when(a + 2 < full)
                def _():
                    icp(a + 2, idx0, si0).start()

                gcp(idx0, rows0, sg0).wait()
                wb(a, rows0)

                @pl.when(a + 2 < full)
                def _():
                    icp(a + 2, idx0, si0).wait()
                    gcp(idx0, rows0, sg0).start()

                @pl.when(a + 3 < full)
                def _():
                    icp(a + 3, idx1, si1).start()

                gcp(idx1, rows1, sg1).wait()
                wb(a + 1, rows1)

            if full % 2 == 1:
                gcp(idx0, rows0, sg0).wait()
                wb(full - 1, rows0)

            @pl.when(wid < rem)
            def _():
                base = (full * 32 + wid) * WIN
                pltpu.sync_copy(eidx_h.at[pl.ds(base, WIN)], idx0)
                gcp(idx0, rows0, sg0).start()
                gcp(idx0, rows0, sg0).wait()
                pltpu.sync_copy(rows0, out_h.at[pl.ds(base, WIN)])

        one_pass(sidx_h, gs_h)
        one_pass(oidx_h, go_h)

    return gather


def _edge_mlp_body(gs_ref, go_ref, pr_ref, ws_ref, wp_ref, wo_ref, b1a_ref,
                   w1b_ref, b1b_ref, *out_refs):
    h = jnp.dot(gs_ref[...].astype(jnp.bfloat16), ws_ref[...],
                preferred_element_type=jnp.float32)
    h += jnp.dot(pr_ref[...].astype(jnp.bfloat16), wp_ref[...],
                 preferred_element_type=jnp.float32)
    h += jnp.dot(go_ref[...].astype(jnp.bfloat16), wo_ref[...],
                 preferred_element_type=jnp.float32)
    h = jnp.maximum(h + b1a_ref[...], 0.0).astype(jnp.bfloat16)
    t = jnp.dot(h, w1b_ref[...], preferred_element_type=jnp.float32)
    t = jnp.maximum(t + b1b_ref[...], 0.0)
    for k in range(9):
        out_refs[k][...] = t[:, 128 * k:128 * (k + 1)]


def _edge_mlp(gs, go, pred, ws, wp, wo, b1a, w1b, b1b):
    ne = gs.shape[0]
    d_out1 = 2 * H + D_OUT_PRED
    wspec = lambda shape: pl.BlockSpec(shape, lambda e: (0, 0))
    espec = lambda w: pl.BlockSpec((BE, w), lambda e: (e, 0))
    return pl.pallas_call(
        _edge_mlp_body,
        grid=(ne // BE,),
        in_specs=[
            espec(D_OBJ), espec(D_OBJ), espec(D_PRED),
            wspec((D_OBJ, H)), wspec((D_PRED, H)), wspec((D_OBJ, H)),
            wspec((1, H)), wspec((H, d_out1)), wspec((1, d_out1)),
        ],
        out_specs=[espec(128) for _ in range(9)],
        out_shape=[jax.ShapeDtypeStruct((ne, 128), jnp.float32)
                   for _ in range(9)],
        compiler_params=pltpu.CompilerParams(
            dimension_semantics=("arbitrary",)),
    )(gs, go, pred, ws, wp, wo, b1a, w1b, b1b)


@functools.lru_cache(maxsize=None)
def _make_scatter(ne):
    nwin = ne // SW

    @functools.partial(
        pl.kernel,
        mesh=_mesh,
        out_type=[jax.ShapeDtypeStruct((NPAD, 128), jnp.float32)
                  for _ in range(4)],
        scratch_types=[
            pltpu.VMEM((SW, 128), jnp.float32),
            pltpu.VMEM((SW, 128), jnp.float32),
            pltpu.VMEM((SW,), jnp.int32),
            pltpu.VMEM((SW,), jnp.int32),
            pltpu.VMEM_SHARED((NPAD, 128), jnp.float32),
            pltpu.SemaphoreType.DMA,
            pltpu.SemaphoreType.DMA,
            pltpu.SemaphoreType.DMA,
            pltpu.SemaphoreType.DMA,
        ],
        compiler_params=_sc_params,
    )
    def scatter(ns0, ns1, ns2, ns3, no0, no1, no2, no3,
                sidx_h, oidx_h, z128_h,
                p0, p1, p2, p3,
                upd0, upd1, idx0, idx1, acc_sh,
                semu0, semu1, semi0, semi1):
        c = lax.axis_index("c")
        s = lax.axis_index("s")

        def scatter_pass(src_h, eidx_h):
            full = nwin // 16
            rem = nwin % 16

            def cps(k, updb, idxb, semu, semi):
                base = (k * 16 + s) * SW
                return (
                    pltpu.make_async_copy(
                        eidx_h.at[pl.ds(base, SW)], idxb, semi),
                    pltpu.make_async_copy(
                        src_h.at[pl.ds(base, SW)], updb, semu),
                )

            def issue(k, updb, idxb, semu, semi):
                for cp in cps(k, updb, idxb, semu, semi):
                    cp.start()

            def wait(k, updb, idxb, semu, semi):
                for cp in cps(k, updb, idxb, semu, semi):
                    cp.wait()

            issue(0, upd0, idx0, semu0, semi0)

            @pl.loop(0, (full - 1) // 2)
            def _(k):
                w = 2 * k
                issue(w + 1, upd1, idx1, semu1, semi1)
                wait(w, upd0, idx0, semu0, semi0)
                pltpu.sync_copy(upd0, acc_sh.at[idx0], add=True)
                issue(w + 2, upd0, idx0, semu0, semi0)
                wait(w + 1, upd1, idx1, semu1, semi1)
                pltpu.sync_copy(upd1, acc_sh.at[idx1], add=True)

            if full % 2 == 1:
                wait(full - 1, upd0, idx0, semu0, semi0)
                pltpu.sync_copy(upd0, acc_sh.at[idx0], add=True)
            else:
                issue(full - 1, upd1, idx1, semu1, semi1)
                wait(full - 2, upd0, idx0, semu0, semi0)
                pltpu.sync_copy(upd0, acc_sh.at[idx0], add=True)
                wait(full - 1, upd1, idx1, semu1, semi1)
                pltpu.sync_copy(upd1, acc_sh.at[idx1], add=True)

            @pl.when(s < rem)
            def _():
                base = (full * 16 + s) * SW
                pltpu.sync_copy(eidx_h.at[pl.ds(base, SW)], idx0)
                pltpu.sync_copy(src_h.at[pl.ds(base, SW)], upd0)
                pltpu.sync_copy(upd0, acc_sh.at[idx0], add=True)

        def one_slab(ns_h, no_h, out_h):
            pltpu.sync_copy(z128_h, upd0)
            for t in range(RPS // SW):
                pltpu.sync_copy(upd0, acc_sh.at[pl.ds(s * RPS + t * SW, SW)])
            plsc.subcore_barrier()
            scatter_pass(ns_h, sidx_h)
            scatter_pass(no_h, oidx_h)
            plsc.subcore_barrier()
            for t in range(RPS // SW):
                rows = pl.ds(s * RPS + t * SW, SW)
                pltpu.sync_copy(acc_sh.at[rows], upd0)
                pltpu.sync_copy(upd0, out_h.at[rows])
            plsc.subcore_barrier()

        @pl.when(c == 0)
        def _():
            one_slab(ns0, no0, p0)
            one_slab(ns1, no1, p1)

        @pl.when(c == 1)
        def _():
            one_slab(ns2, no2, p2)
            one_slab(ns3, no3, p3)

    return scatter


@functools.partial(
    pl.kernel,
    mesh=_mesh,
    out_type=[jax.ShapeDtypeStruct((NPAD // 128, 128), jnp.float32)
              for _ in range(2)],
    scratch_types=[
        pltpu.VMEM((CCH,), jnp.int32),
        pltpu.VMEM((NPAD // 128, 128), jnp.float32),
        pltpu.VMEM((NPAD // 128,), jnp.int32),
        pltpu.VMEM_SHARED((NPAD // 128, 128), jnp.float32),
    ],
    compiler_params=_sc_params,
)
def _sc_counts(sidx_h, oidx_h, c0_out, c1_out,
               cidx_v, hist_v, rowidx_v, csh):
    c = lax.axis_index("c")
    s = lax.axis_index("s")
    nrows_h = NPAD // 128
    zero16 = jnp.zeros((16,), jnp.float32)
    one16 = jnp.full((16,), 1.0, jnp.float32)

    def hist_pass(eidx_h, out_h):
        @pl.loop(0, nrows_h)
        def _(r):
            for j in range(8):
                hist_v[r, pl.ds(j * 16, 16)] = zero16

        @pl.when(s == 0)
        def _():
            pltpu.sync_copy(hist_v, csh)

        @pl.loop(0, 5)
        def _(t):
            rowidx_v[pl.ds(t * 16, 16)] = lax.iota(jnp.int32, 16) + t * 16

        @pl.loop(0, E // 16 // CCH)
        def _(h):
            base = s * (E // 16) + h * CCH
            pltpu.sync_copy(eidx_h.at[pl.ds(base, CCH)], cidx_v)

            @pl.loop(0, CCH // 16)
            def _(t):
                v = cidx_v[pl.ds(t * 16, 16)]
                r = lax.shift_right_logical(v, 7)
                col = lax.bitwise_and(v, 127)
                plsc.addupdate_scatter(hist_v, [r, col], one16)

        plsc.subcore_barrier()
        pltpu.sync_copy(hist_v, csh.at[rowidx_v], add=True)
        plsc.subcore_barrier()

        @pl.when(s == 0)
        def _():
            pltpu.sync_copy(csh, hist_v)
            pltpu.sync_copy(hist_v, out_h)

    @pl.when(c == 0)
    def _():
        hist_pass(sidx_h, c0_out)

    @pl.when(c == 1)
    def _():
        hist_pass(oidx_h, c1_out)


def _node_mlp_body(*refs):
    nparts = len(CHUNKS)
    p = refs[:4 * nparts]
    c0_ref, c1_ref, w2a_ref, b2a_ref, w2b_ref, b2b_ref, out_ref = (
        refs[4 * nparts:])
    cnt = jnp.maximum(c0_ref[...] + c1_ref[...], 1.0)
    cols = []
    for j in range(4):
        acc = p[j][...]
        for kp in range(1, nparts):
            acc = acc + p[kp * 4 + j][...]
        cols.append(acc)
    x = jnp.concatenate(cols, axis=1) / cnt
    h = jnp.dot(x.astype(jnp.bfloat16), w2a_ref[...],
                preferred_element_type=jnp.float32)
    h = jnp.maximum(h + b2a_ref[...], 0.0).astype(jnp.bfloat16)
    o = jnp.dot(h, w2b_ref[...], preferred_element_type=jnp.float32)
    out_ref[...] = jnp.maximum(o + b2b_ref[...], 0.0)


def _node_mlp(parts, c0, c1, w2a, b2a, w2b, b2b):
    nparts = len(CHUNKS)
    wspec = lambda shape: pl.BlockSpec(shape, lambda n: (0, 0))
    nspec = lambda w: pl.BlockSpec((BN, w), lambda n: (n, 0))
    flat = [a for part in parts for a in part]
    return pl.pallas_call(
        _node_mlp_body,
        grid=(NPAD // BN,),
        in_specs=[nspec(128)] * (4 * nparts) + [
            pl.BlockSpec((BN, 1), lambda n: (n, 0)),
            pl.BlockSpec((BN, 1), lambda n: (n, 0)),
            wspec((H, H)), wspec((1, H)), wspec((H, D_OUT)),
            wspec((1, D_OUT)),
        ],
        out_specs=nspec(D_OUT),
        out_shape=jax.ShapeDtypeStruct((NPAD, D_OUT), jnp.float32),
        compiler_params=pltpu.CompilerParams(
            dimension_semantics=("arbitrary",)),
    )(*flat, c0.reshape(NPAD, 1), c1.reshape(NPAD, 1),
      w2a, b2a, w2b, b2b)


def kernel(obj_vecs, pred_vecs, edges, nodes_per_img,
           W1a, b1a, W1b, b1b, W2a, b2a, W2b, b2b):
    del nodes_per_img
    s_idx = edges[:, 0]
    o_idx = edges[:, 1]

    ws = W1a[:D_OBJ].astype(jnp.bfloat16)
    wp = W1a[D_OBJ:D_OBJ + D_PRED].astype(jnp.bfloat16)
    wo = W1a[D_OBJ + D_PRED:].astype(jnp.bfloat16)
    w1b = W1b.astype(jnp.bfloat16)
    w2a = W2a.astype(jnp.bfloat16)
    w2b = W2b.astype(jnp.bfloat16)
    b1a2 = b1a.reshape(1, H)
    b1b2 = b1b.reshape(1, 2 * H + D_OUT_PRED)
    z128 = jnp.zeros((SW, 128), jnp.float32)


    c0, c1 = _sc_counts(s_idx, o_idx)

    new_p_parts = []
    pooled_parts = []
    lo = 0
    for ne in CHUNKS:
        sl = slice(lo, lo + ne)
        lo += ne
        gs, go = _make_gather(ne)(obj_vecs, s_idx[sl], o_idx[sl])
        outs = _edge_mlp(gs, go, pred_vecs[sl], ws, wp, wo, b1a2, w1b, b1b2)
        ns0, ns1, ns2, ns3, np_c, no0, no1, no2, no3 = outs
        new_p_parts.append(np_c)
        pooled_parts.append(_make_scatter(ne)(
            ns0, ns1, ns2, ns3, no0, no1, no2, no3,
            s_idx[sl], o_idx[sl], z128))

    new_obj = _node_mlp(pooled_parts, c0, c1,
                        w2a, b2a.reshape(1, H), w2b, b2b.reshape(1, D_OUT))
    new_p = jnp.concatenate(new_p_parts, axis=0)
    return (new_obj[:N], new_p)


# speedup vs baseline: 1.0212x; 1.0212x over previous
import dataclasses
import functools

import jax
import jax.numpy as jnp
from jax import lax
from jax.experimental import pallas as pl
from jax.experimental.pallas import tpu as pltpu
from jax.experimental.pallas import tpu_sc as plsc

N = 10000
E = 160000
D_OBJ = 256
D_PRED = 128
H = 512
D_OUT = 256
D_OUT_PRED = 128

NPAD = 10240
RPS = NPAD // 16
WIN = 128
BE = 640
BN = 512
SW = 128
CCH = 2000
CHUNKS = (80640, 79360)

_mesh = plsc.VectorSubcoreMesh(core_axis_name="c", subcore_axis_name="s")

_sc_params = pltpu.CompilerParams()
if "needs_layout_passes" in pltpu.CompilerParams.__dataclass_fields__:
    _sc_params = dataclasses.replace(_sc_params, needs_layout_passes=False)


@functools.lru_cache(maxsize=None)
def _make_gather(ne):
    nwin = ne // WIN

    @functools.partial(
        pl.kernel,
        mesh=_mesh,
        out_type=[
            jax.ShapeDtypeStruct((ne, D_OBJ), jnp.float32),
            jax.ShapeDtypeStruct((ne, D_OBJ), jnp.float32),
        ],
        scratch_types=[
            pltpu.VMEM((WIN,), jnp.int32),
            pltpu.VMEM((WIN,), jnp.int32),
            pltpu.VMEM((WIN, D_OBJ), jnp.float32),
            pltpu.VMEM((WIN, D_OBJ), jnp.float32),
            pltpu.SemaphoreType.DMA,
            pltpu.SemaphoreType.DMA,
            pltpu.SemaphoreType.DMA,
            pltpu.SemaphoreType.DMA,
        ],
    )
    def gather(obj_h, sidx_h, oidx_h, gs_h, go_h,
               idx0, idx1, rows0, rows1, si0, si1, sg0, sg1):
        wid = lax.axis_index("s") * 2 + lax.axis_index("c")
        full = nwin // 32
        rem = nwin % 32

        def one_pass(eidx_h, out_h):
            def icp(k, idxb, sem):
                return pltpu.make_async_copy(
                    eidx_h.at[pl.ds((k * 32 + wid) * WIN, WIN)], idxb, sem)

            def gcp(idxb, rowsb, sem):
                return pltpu.make_async_copy(obj_h.at[idxb], rowsb, sem)

            def wb(k, rowsb):
                pltpu.sync_copy(
                    rowsb, out_h.at[pl.ds((k * 32 + wid) * WIN, WIN)])

            icp(0, idx0, si0).start()
            icp(0, idx0, si0).wait()
            gcp(idx0, rows0, sg0).start()
            icp(1, idx1, si1).start()

            @pl.loop(0, full // 2)
            def _(t):
                a = 2 * t
                icp(a + 1, idx1, si1).wait()
                gcp(idx1, rows1, sg1).start()

                @pl.when(a + 2 < full)
                def _():
                    icp(a + 2, idx0, si0).start()

                gcp(idx0, rows0, sg0).wait()
                wb(a, rows0)

                @pl.when(a + 2 < full)
                def _():
                    icp(a + 2, idx0, si0).wait()
                    gcp(idx0, rows0, sg0).start()

                @pl.when(a + 3 < full)
                def _():
                    icp(a + 3, idx1, si1).start()

                gcp(idx1, rows1, sg1).wait()
                wb(a + 1, rows1)

            if full % 2 == 1:
                gcp(idx0, rows0, sg0).wait()
                wb(full - 1, rows0)

            @pl.when(wid < rem)
            def _():
                base = (full * 32 + wid) * WIN
                pltpu.sync_copy(eidx_h.at[pl.ds(base, WIN)], idx0)
                gcp(idx0, rows0, sg0).start()
                gcp(idx0, rows0, sg0).wait()
                pltpu.sync_copy(rows0, out_h.at[pl.ds(base, WIN)])

        one_pass(sidx_h, gs_h)
        one_pass(oidx_h, go_h)

    return gather


def _edge_mlp_body(gs_ref, go_ref, pr_ref, ws_ref, wp_ref, wo_ref, b1a_ref,
                   w1b_ref, b1b_ref, *out_refs):
    h = jnp.dot(gs_ref[...].astype(jnp.bfloat16), ws_ref[...],
                preferred_element_type=jnp.float32)
    h += jnp.dot(pr_ref[...].astype(jnp.bfloat16), wp_ref[...],
                 preferred_element_type=jnp.float32)
    h += jnp.dot(go_ref[...].astype(jnp.bfloat16), wo_ref[...],
                 preferred_element_type=jnp.float32)
    h = jnp.maximum(h + b1a_ref[...], 0.0).astype(jnp.bfloat16)
    t = jnp.dot(h, w1b_ref[...], preferred_element_type=jnp.float32)
    t = jnp.maximum(t + b1b_ref[...], 0.0)
    for k in range(9):
        out_refs[k][...] = t[:, 128 * k:128 * (k + 1)]


def _edge_mlp(gs, go, pred, ws, wp, wo, b1a, w1b, b1b):
    ne = gs.shape[0]
    d_out1 = 2 * H + D_OUT_PRED
    wspec = lambda shape: pl.BlockSpec(shape, lambda e: (0, 0))
    espec = lambda w: pl.BlockSpec((BE, w), lambda e: (e, 0))
    return pl.pallas_call(
        _edge_mlp_body,
        grid=(ne // BE,),
        in_specs=[
            espec(D_OBJ), espec(D_OBJ), espec(D_PRED),
            wspec((D_OBJ, H)), wspec((D_PRED, H)), wspec((D_OBJ, H)),
            wspec((1, H)), wspec((H, d_out1)), wspec((1, d_out1)),
        ],
        out_specs=[espec(128) for _ in range(9)],
        out_shape=[jax.ShapeDtypeStruct((ne, 128), jnp.float32)
                   for _ in range(9)],
        compiler_params=pltpu.CompilerParams(
            dimension_semantics=("parallel",)),
    )(gs, go, pred, ws, wp, wo, b1a, w1b, b1b)


@functools.lru_cache(maxsize=None)
def _make_scatter(ne):
    nwin = ne // SW

    @functools.partial(
        pl.kernel,
        mesh=_mesh,
        out_type=[jax.ShapeDtypeStruct((NPAD, 128), jnp.float32)
                  for _ in range(4)],
        scratch_types=[
            pltpu.VMEM((SW, 128), jnp.float32),
            pltpu.VMEM((SW, 128), jnp.float32),
            pltpu.VMEM((SW,), jnp.int32),
            pltpu.VMEM((SW,), jnp.int32),
            pltpu.VMEM_SHARED((NPAD, 128), jnp.float32),
            pltpu.SemaphoreType.DMA,
            pltpu.SemaphoreType.DMA,
            pltpu.SemaphoreType.DMA,
            pltpu.SemaphoreType.DMA,
        ],
        compiler_params=_sc_params,
    )
    def scatter(ns0, ns1, ns2, ns3, no0, no1, no2, no3,
                sidx_h, oidx_h, z128_h,
                p0, p1, p2, p3,
                upd0, upd1, idx0, idx1, acc_sh,
                semu0, semu1, semi0, semi1):
        c = lax.axis_index("c")
        s = lax.axis_index("s")

        def scatter_pass(src_h, eidx_h):
            full = nwin // 16
            rem = nwin % 16

            def cps(k, updb, idxb, semu, semi):
                base = (k * 16 + s) * SW
                return (
                    pltpu.make_async_copy(
                        eidx_h.at[pl.ds(base, SW)], idxb, semi),
                    pltpu.make_async_copy(
                        src_h.at[pl.ds(base, SW)], updb, semu),
                )

            def issue(k, updb, idxb, semu, semi):
                for cp in cps(k, updb, idxb, semu, semi):
                    cp.start()

            def wait(k, updb, idxb, semu, semi):
                for cp in cps(k, updb, idxb, semu, semi):
                    cp.wait()

            issue(0, upd0, idx0, semu0, semi0)

            @pl.loop(0, (full - 1) // 2)
            def _(k):
                w = 2 * k
                issue(w + 1, upd1, idx1, semu1, semi1)
                wait(w, upd0, idx0, semu0, semi0)
                pltpu.sync_copy(upd0, acc_sh.at[idx0], add=True)
                issue(w + 2, upd0, idx0, semu0, semi0)
                wait(w + 1, upd1, idx1, semu1, semi1)
                pltpu.sync_copy(upd1, acc_sh.at[idx1], add=True)

            if full % 2 == 1:
                wait(full - 1, upd0, idx0, semu0, semi0)
                pltpu.sync_copy(upd0, acc_sh.at[idx0], add=True)
            else:
                issue(full - 1, upd1, idx1, semu1, semi1)
                wait(full - 2, upd0, idx0, semu0, semi0)
                pltpu.sync_copy(upd0, acc_sh.at[idx0], add=True)
                wait(full - 1, upd1, idx1, semu1, semi1)
                pltpu.sync_copy(upd1, acc_sh.at[idx1], add=True)

            @pl.when(s < rem)
            def _():
                base = (full * 16 + s) * SW
                pltpu.sync_copy(eidx_h.at[pl.ds(base, SW)], idx0)
                pltpu.sync_copy(src_h.at[pl.ds(base, SW)], upd0)
                pltpu.sync_copy(upd0, acc_sh.at[idx0], add=True)

        def one_slab(ns_h, no_h, out_h):
            pltpu.sync_copy(z128_h, upd0)
            for t in range(RPS // SW):
                pltpu.sync_copy(upd0, acc_sh.at[pl.ds(s * RPS + t * SW, SW)])
            plsc.subcore_barrier()
            scatter_pass(ns_h, sidx_h)
            scatter_pass(no_h, oidx_h)
            plsc.subcore_barrier()
            for t in range(RPS // SW):
                rows = pl.ds(s * RPS + t * SW, SW)
                pltpu.sync_copy(acc_sh.at[rows], upd0)
                pltpu.sync_copy(upd0, out_h.at[rows])
            plsc.subcore_barrier()

        @pl.when(c == 0)
        def _():
            one_slab(ns0, no0, p0)
            one_slab(ns1, no1, p1)

        @pl.when(c == 1)
        def _():
            one_slab(ns2, no2, p2)
            one_slab(ns3, no3, p3)

    return scatter


@functools.partial(
    pl.kernel,
    mesh=_mesh,
    out_type=[jax.ShapeDtypeStruct((NPAD // 128, 128), jnp.float32)
              for _ in range(2)],
    scratch_types=[
        pltpu.VMEM((CCH,), jnp.int32),
        pltpu.VMEM((NPAD // 128, 128), jnp.float32),
        pltpu.VMEM((NPAD // 128,), jnp.int32),
        pltpu.VMEM_SHARED((NPAD // 128, 128), jnp.float32),
    ],
    compiler_params=_sc_params,
)
def _sc_counts(sidx_h, oidx_h, c0_out, c1_out,
               cidx_v, hist_v, rowidx_v, csh):
    c = lax.axis_index("c")
    s = lax.axis_index("s")
    nrows_h = NPAD // 128
    zero16 = jnp.zeros((16,), jnp.float32)
    one16 = jnp.full((16,), 1.0, jnp.float32)

    def hist_pass(eidx_h, out_h):
        @pl.loop(0, nrows_h)
        def _(r):
            for j in range(8):
                hist_v[r, pl.ds(j * 16, 16)] = zero16

        @pl.when(s == 0)
        def _():
            pltpu.sync_copy(hist_v, csh)

        @pl.loop(0, 5)
        def _(t):
            rowidx_v[pl.ds(t * 16, 16)] = lax.iota(jnp.int32, 16) + t * 16

        @pl.loop(0, E // 16 // CCH)
        def _(h):
            base = s * (E // 16) + h * CCH
            pltpu.sync_copy(eidx_h.at[pl.ds(base, CCH)], cidx_v)

            @pl.loop(0, CCH // 16)
            def _(t):
                v = cidx_v[pl.ds(t * 16, 16)]
                r = lax.shift_right_logical(v, 7)
                col = lax.bitwise_and(v, 127)
                plsc.addupdate_scatter(hist_v, [r, col], one16)

        plsc.subcore_barrier()
        pltpu.sync_copy(hist_v, csh.at[rowidx_v], add=True)
        plsc.subcore_barrier()

        @pl.when(s == 0)
        def _():
            pltpu.sync_copy(csh, hist_v)
            pltpu.sync_copy(hist_v, out_h)

    @pl.when(c == 0)
    def _():
        hist_pass(sidx_h, c0_out)

    @pl.when(c == 1)
    def _():
        hist_pass(oidx_h, c1_out)


def _node_mlp_body(*refs):
    nparts = len(CHUNKS)
    p = refs[:4 * nparts]
    c0_ref, c1_ref, w2a_ref, b2a_ref, w2b_ref, b2b_ref, out_ref = (
        refs[4 * nparts:])
    cnt = jnp.maximum(c0_ref[...] + c1_ref[...], 1.0)
    cols = []
    for j in range(4):
        acc = p[j][...]
        for kp in range(1, nparts):
            acc = acc + p[kp * 4 + j][...]
        cols.append(acc)
    x = jnp.concatenate(cols, axis=1) / cnt
    h = jnp.dot(x.astype(jnp.bfloat16), w2a_ref[...],
                preferred_element_type=jnp.float32)
    h = jnp.maximum(h + b2a_ref[...], 0.0).astype(jnp.bfloat16)
    o = jnp.dot(h, w2b_ref[...], preferred_element_type=jnp.float32)
    out_ref[...] = jnp.maximum(o + b2b_ref[...], 0.0)


def _node_mlp(parts, c0, c1, w2a, b2a, w2b, b2b):
    nparts = len(CHUNKS)
    wspec = lambda shape: pl.BlockSpec(shape, lambda n: (0, 0))
    nspec = lambda w: pl.BlockSpec((BN, w), lambda n: (n, 0))
    flat = [a for part in parts for a in part]
    return pl.pallas_call(
        _node_mlp_body,
        grid=(NPAD // BN,),
        in_specs=[nspec(128)] * (4 * nparts) + [
            pl.BlockSpec((BN, 1), lambda n: (n, 0)),
            pl.BlockSpec((BN, 1), lambda n: (n, 0)),
            wspec((H, H)), wspec((1, H)), wspec((H, D_OUT)),
            wspec((1, D_OUT)),
        ],
        out_specs=nspec(D_OUT),
        out_shape=jax.ShapeDtypeStruct((NPAD, D_OUT), jnp.float32),
        compiler_params=pltpu.CompilerParams(
            dimension_semantics=("parallel",)),
    )(*flat, c0.reshape(NPAD, 1), c1.reshape(NPAD, 1),
      w2a, b2a, w2b, b2b)


def kernel(obj_vecs, pred_vecs, edges, nodes_per_img,
           W1a, b1a, W1b, b1b, W2a, b2a, W2b, b2b):
    del nodes_per_img
    s_idx = edges[:, 0]
    o_idx = edges[:, 1]

    ws = W1a[:D_OBJ].astype(jnp.bfloat16)
    wp = W1a[D_OBJ:D_OBJ + D_PRED].astype(jnp.bfloat16)
    wo = W1a[D_OBJ + D_PRED:].astype(jnp.bfloat16)
    w1b = W1b.astype(jnp.bfloat16)
    w2a = W2a.astype(jnp.bfloat16)
    w2b = W2b.astype(jnp.bfloat16)
    b1a2 = b1a.reshape(1, H)
    b1b2 = b1b.reshape(1, 2 * H + D_OUT_PRED)
    z128 = jnp.zeros((SW, 128), jnp.float32)


    c0, c1 = _sc_counts(s_idx, o_idx)

    new_p_parts = []
    pooled_parts = []
    lo = 0
    for ne in CHUNKS:
        sl = slice(lo, lo + ne)
        lo += ne
        gs, go = _make_gather(ne)(obj_vecs, s_idx[sl], o_idx[sl])
        outs = _edge_mlp(gs, go, pred_vecs[sl], ws, wp, wo, b1a2, w1b, b1b2)
        ns0, ns1, ns2, ns3, np_c, no0, no1, no2, no3 = outs
        new_p_parts.append(np_c)
        pooled_parts.append(_make_scatter(ne)(
            ns0, ns1, ns2, ns3, no0, no1, no2, no3,
            s_idx[sl], o_idx[sl], z128))

    new_obj = _node_mlp(pooled_parts, c0, c1,
                        w2a, b2a.reshape(1, H), w2b, b2b.reshape(1, D_OUT))
    new_p = jnp.concatenate(new_p_parts, axis=0)
    return (new_obj[:N], new_p)
